# Initial kernel scaffold; baseline (speedup 1.0000x reference)
import jax
import jax.numpy as jnp
from jax.experimental import pallas as pl


def kernel(inputs, gate_w, expert_w, expert_b):
    raise NotImplementedError("write your pallas kernel here")


# speedup vs baseline: 1.0716x; 1.0716x over previous
import functools

import jax
import jax.numpy as jnp
from jax.experimental import pallas as pl
from jax.experimental.pallas import tpu as pltpu


def _fused_moe_body(x_ref, gw_ref, ew_ref, eb_ref, out_ref, *, n_e):
    e = pl.program_id(2)
    x = x_ref[...]
    bt = x.shape[0]
    logits = jax.lax.dot_general(
        x, gw_ref[...], (((1,), (1,)), ((), ())),
        preferred_element_type=jnp.float32)
    i8 = jax.lax.broadcasted_iota(jnp.int32, (bt, n_e), 1)
    v1 = jnp.max(logits, axis=1, keepdims=True)
    i1 = jnp.min(jnp.where(logits == v1, i8, n_e), axis=1, keepdims=True)
    l2 = jnp.where(i8 == i1, -jnp.inf, logits)
    v2 = jnp.max(l2, axis=1, keepdims=True)
    i2 = jnp.min(jnp.where(l2 == v2, i8, n_e), axis=1, keepdims=True)
    t = jnp.exp(v2 - v1)
    p1 = 1.0 / (1.0 + t)
    p2 = t / (1.0 + t)
    w_e = jnp.where(i1 == e, p1, 0.0) + jnp.where(i2 == e, p2, 0.0)
    y = jax.lax.dot_general(
        x, ew_ref[0], (((1,), (1,)), ((), ())),
        preferred_element_type=jnp.float32) + eb_ref[0]
    val = w_e * y
    @pl.when(e == 0)
    def _():
        out_ref[...] = val
    @pl.when(e != 0)
    def _():
        out_ref[...] += val


def kernel(inputs, gate_w, expert_w, expert_b):
    batch_shape = inputs.shape[:-1]
    d_in = inputs.shape[-1]
    x = inputs.reshape(-1, d_in)
    t_tot = x.shape[0]
    n_e, d_out = expert_b.shape
    bt = min(1024, t_tot)
    bdo = min(1024, d_out)
    grid = (t_tot // bt, d_out // bdo, n_e)
    out = pl.pallas_call(
        functools.partial(_fused_moe_body, n_e=n_e),
        grid=grid,
        in_specs=[
            pl.BlockSpec((bt, d_in), lambda t, do, e: (t, 0)),
            pl.BlockSpec((n_e, d_in), lambda t, do, e: (0, 0)),
            pl.BlockSpec((1, bdo, d_in), lambda t, do, e: (e, do, 0)),
            pl.BlockSpec((1, 1, bdo), lambda t, do, e: (e, 0, do)),
        ],
        out_specs=pl.BlockSpec((bt, bdo), lambda t, do, e: (t, do)),
        out_shape=jax.ShapeDtypeStruct((t_tot, d_out), jnp.float32),
        compiler_params=pltpu.CompilerParams(
            dimension_semantics=("arbitrary", "arbitrary", "arbitrary")),
    )(x, gate_w, expert_w, expert_b.reshape(n_e, 1, d_out))
    return out.reshape(*batch_shape, d_out)
